# baseline (speedup 1.0000x reference)
import functools
import math

import jax
import jax.numpy as jnp
from jax import lax
from jax.experimental import pallas as pl
from jax.experimental.pallas import tpu as pltpu
from jax.experimental.pallas import tpu_sc as plsc

D_MODEL = 64
SCALE = math.sqrt(D_MODEL)
LANES = 16
NUM_CORES = 2
NUM_SUBCORES = 16
NUM_WORKERS = NUM_CORES * NUM_SUBCORES
NBUF = 4
HIST_PAD = 56


@functools.lru_cache(maxsize=None)
def _make_kernel(batch: int, hist: int):
    assert batch % NUM_WORKERS == 0
    b_per_w = batch // NUM_WORKERS
    assert b_per_w % NBUF == 0
    mesh = plsc.VectorSubcoreMesh(core_axis_name="c", subcore_axis_name="s")

    @functools.partial(
        pl.kernel,
        mesh=mesh,
        out_type=jax.ShapeDtypeStruct((batch, hist, D_MODEL), jnp.float32),
        scratch_types=[
            pltpu.VMEM((b_per_w, 128), jnp.int32),
            pltpu.VMEM((HIST_PAD, D_MODEL), jnp.float32),
            pltpu.VMEM((HIST_PAD, D_MODEL), jnp.float32),
            pltpu.VMEM((HIST_PAD, D_MODEL), jnp.float32),
            pltpu.VMEM((HIST_PAD, D_MODEL), jnp.float32),
            pltpu.SemaphoreType.DMA,
            pltpu.SemaphoreType.DMA,
            pltpu.SemaphoreType.DMA,
            pltpu.SemaphoreType.DMA,
        ],
        compiler_params=pltpu.CompilerParams(use_tc_tiling_on_sc=False),
    )
    def k(x_hbm, table_hbm, out_hbm, idx_v, b0, b1, b2, b3, s0, s1, s2, s3):
        wid = lax.axis_index("s") * NUM_CORES + lax.axis_index("c")
        base = wid * b_per_w
        pltpu.sync_copy(x_hbm.at[pl.ds(base, b_per_w)], idx_v)

        bufs = (b0, b1, b2, b3)
        sems = (s0, s1, s2, s3)

        def idx_list(j):
            return idx_v.at[j, pl.ds(0, HIST_PAD)]

        for j in range(NBUF - 1):
            pltpu.async_copy(table_hbm.at[idx_list(j)], bufs[j], sems[j])

        def outer(g, _):
            for u in range(NBUF):
                j = g * NBUF + u
                buf = bufs[u]
                sem = sems[u]
                nxt = (u + NBUF - 1) % NBUF

                @pl.when(j + NBUF - 1 < b_per_w)
                def _():
                    pltpu.async_copy(
                        table_hbm.at[idx_list(j + NBUF - 1)], bufs[nxt], sems[nxt]
                    )

                pltpu.make_async_copy(table_hbm.at[idx_list(j)], buf, sem).wait()

                def scale_row(r, _):
                    for c in range(D_MODEL // LANES):
                        sl = pl.ds(c * LANES, LANES)
                        buf[r, sl] = buf[r, sl] * SCALE
                    return 0

                lax.fori_loop(0, hist, scale_row, 0)
                pltpu.sync_copy(buf.at[pl.ds(0, hist)], out_hbm.at[base + j])
            return 0

        lax.fori_loop(0, b_per_w // NBUF, outer, 0)

    return k


def kernel(x, table):
    batch, hist = x.shape
    xp = jnp.pad(x.astype(jnp.int32), ((0, 0), (0, 128 - hist)))
    return _make_kernel(batch, hist)(xp, table)


# speedup vs baseline: 2.5963x; 2.5963x over previous
import functools
import math

import jax
import jax.numpy as jnp
from jax import lax
from jax.experimental import pallas as pl
from jax.experimental.pallas import tpu as pltpu
from jax.experimental.pallas import tpu_sc as plsc

D_MODEL = 64
SCALE = math.sqrt(D_MODEL)
LANES = 16
CHUNK = 128
NUM_CORES = 2
NUM_SUBCORES = 16
NUM_WORKERS = NUM_CORES * NUM_SUBCORES


@functools.lru_cache(maxsize=None)
def _make_kernel(B: int):
    assert B % (NUM_WORKERS * CHUNK) == 0
    b_per_w = B // NUM_WORKERS
    nch = b_per_w // CHUNK
    mesh = plsc.VectorSubcoreMesh(core_axis_name="c", subcore_axis_name="s")

    @functools.partial(
        pl.kernel,
        mesh=mesh,
        out_type=jax.ShapeDtypeStruct((B, D_MODEL), jnp.float32),
        scratch_types=[
            pltpu.VMEM((CHUNK,), jnp.int32),
            pltpu.VMEM((CHUNK,), jnp.int32),
            pltpu.VMEM((CHUNK, D_MODEL), jnp.float32),
            pltpu.VMEM((CHUNK, D_MODEL), jnp.float32),
            pltpu.SemaphoreType.DMA,
            pltpu.SemaphoreType.DMA,
            pltpu.SemaphoreType.DMA,
            pltpu.SemaphoreType.DMA,
        ],
        compiler_params=pltpu.CompilerParams(use_tc_tiling_on_sc=False),
    )
    def k(x_hbm, table_hbm, out_hbm, i0, i1, b0, b1, si0, si1, sg0, sg1):
        wid = lax.axis_index("s") * NUM_CORES + lax.axis_index("c")
        base = wid * b_per_w

        idxs = (i0, i1)
        bufs = (b0, b1)
        isems = (si0, si1)
        gsems = (sg0, sg1)

        def fetch_idx(j, u):
            pltpu.async_copy(
                x_hbm.at[pl.ds(base + j * CHUNK, CHUNK)], idxs[u], isems[u]
            )

        def wait_idx(j, u):
            pltpu.make_async_copy(
                x_hbm.at[pl.ds(base + j * CHUNK, CHUNK)], idxs[u], isems[u]
            ).wait()

        def fire_gather(u):
            pltpu.async_copy(table_hbm.at[idxs[u]], bufs[u], gsems[u])

        def wait_gather(u):
            pltpu.make_async_copy(table_hbm.at[idxs[u]], bufs[u], gsems[u]).wait()

        fetch_idx(0, 0)
        wait_idx(0, 0)
        fire_gather(0)
        fetch_idx(1, 1)

        def outer(g, _):
            for u in range(2):
                j = g * 2 + u
                o = 1 - u

                @pl.when(j + 1 < nch)
                def _():
                    wait_idx(j + 1, o)
                    fire_gather(o)

                wait_gather(u)

                @pl.when(j + 2 < nch)
                def _():
                    fetch_idx(j + 2, u)

                buf = bufs[u]

                def scale_row(r, _):
                    for c in range(D_MODEL // LANES):
                        sl = pl.ds(c * LANES, LANES)
                        buf[r, sl] = buf[r, sl] * SCALE
                    return 0

                lax.fori_loop(0, CHUNK, scale_row, 0)
                pltpu.sync_copy(buf, out_hbm.at[pl.ds(base + j * CHUNK, CHUNK)])
            return 0

        lax.fori_loop(0, nch // 2, outer, 0)

    return k


def kernel(x, table):
    batch, hist = x.shape
    B = batch * hist
    xf = x.reshape(-1).astype(jnp.int32)
    out = _make_kernel(B)(xf, table)
    return out.reshape(batch, hist, D_MODEL)
